# baseline (speedup 1.0000x reference)
import jax
import jax.numpy as jnp
from jax import lax
from jax.experimental import pallas as pl
from jax.experimental.pallas import tpu as pltpu
from jax.experimental.pallas import tpu_sc as plsc

N = 10000
E = 320000
ANN = 64
D = 128
T = 4
STEPS = 5
CLS = 10

NP = 10240
NSUB = 16
NW = 32
RT = NP // NSUB
CHUNK = 128
HC = 40
CT = 2 * HC
EW = CT * CHUNK
EP = NW * EW
BR = 640


def _sc_body(trans_hbm, gidx_hbm, dst_hbm, out0_hbm, out1_hbm,
             gidx_v, dst_v, rows_a, rows_b, acc, sem_a, sem_b, sem_sa,
             sem_sb):
    c = lax.axis_index("c")
    s = lax.axis_index("s")
    wid = c * NSUB + s

    def zrow(r, carry):
        for k in range(D // 16):
            rows_a[r, pl.ds(k * 16, 16)] = jnp.zeros((16,), jnp.float32)
        return carry
    lax.fori_loop(0, CHUNK, zrow, 0)
    for i in range(RT // CHUNK):
        pltpu.sync_copy(rows_a, acc.at[pl.ds(s * RT + i * CHUNK, CHUNK)])
    plsc.subcore_barrier()

    def scat(rows, dv_base, sem):
        pltpu.async_copy(rows.at[pl.ds(0, CHUNK // 2)],
                         acc.at[dst_v.at[dv_base]], sem, add=True)
        pltpu.async_copy(rows.at[pl.ds(CHUNK // 2, CHUNK // 2)],
                         acc.at[dst_v.at[dv_base + 1]], sem, add=True)

    def wait_scat(rows, dv_base, sem):
        pltpu.make_async_copy(rows.at[pl.ds(0, CHUNK // 2)],
                              acc.at[dst_v.at[dv_base]], sem).wait()
        pltpu.make_async_copy(rows.at[pl.ds(CHUNK // 2, CHUNK // 2)],
                              acc.at[dst_v.at[dv_base + 1]], sem).wait()

    def half(hb, carry):
        pltpu.sync_copy(gidx_hbm.at[wid, hb], gidx_v)
        pltpu.sync_copy(dst_hbm.at[wid, hb], dst_v)
        pltpu.async_copy(trans_hbm.at[gidx_v.at[0]], rows_a, sem_a)
        pltpu.async_copy(trans_hbm.at[gidx_v.at[1]], rows_b, sem_b)

        def pair(jj, carry2):
            j0 = jj * 2
            pltpu.make_async_copy(trans_hbm.at[gidx_v.at[j0]], rows_a,
                                  sem_a).wait()
            scat(rows_a, j0 * 2, sem_sa)
            pltpu.make_async_copy(trans_hbm.at[gidx_v.at[j0 + 1]], rows_b,
                                  sem_b).wait()
            scat(rows_b, j0 * 2 + 2, sem_sb)
            wait_scat(rows_a, j0 * 2, sem_sa)
            pltpu.async_copy(trans_hbm.at[gidx_v.at[j0 + 2]], rows_a, sem_a)
            wait_scat(rows_b, j0 * 2 + 2, sem_sb)
            pltpu.async_copy(trans_hbm.at[gidx_v.at[j0 + 3]], rows_b, sem_b)
            return carry2
        lax.fori_loop(0, HC // 2 - 1, pair, 0)

        j0 = HC - 2
        pltpu.make_async_copy(trans_hbm.at[gidx_v.at[j0]], rows_a,
                              sem_a).wait()
        scat(rows_a, j0 * 2, sem_sa)
        pltpu.make_async_copy(trans_hbm.at[gidx_v.at[j0 + 1]], rows_b,
                              sem_b).wait()
        scat(rows_b, j0 * 2 + 2, sem_sb)
        wait_scat(rows_a, j0 * 2, sem_sa)
        wait_scat(rows_b, j0 * 2 + 2, sem_sb)
        return carry
    lax.fori_loop(0, 2, half, 0)
    plsc.subcore_barrier()

    @pl.when(c == 0)
    def _():
        pltpu.sync_copy(acc.at[pl.ds(s * RT, RT)], out0_hbm.at[pl.ds(s * RT, RT)])

    @pl.when(c == 1)
    def _():
        pltpu.sync_copy(acc.at[pl.ds(s * RT, RT)], out1_hbm.at[pl.ds(s * RT, RT)])


_sc_scatter = pl.kernel(
    _sc_body,
    out_type=(jax.ShapeDtypeStruct((NP, D), jnp.float32),
              jax.ShapeDtypeStruct((NP, D), jnp.float32)),
    mesh=plsc.VectorSubcoreMesh(core_axis_name="c", subcore_axis_name="s"),
    scratch_types=[
        pltpu.VMEM((HC, CHUNK), jnp.int32),
        pltpu.VMEM((2 * HC, CHUNK // 2), jnp.int32),
        pltpu.VMEM((CHUNK, D), jnp.float32),
        pltpu.VMEM((CHUNK, D), jnp.float32),
        pltpu.VMEM_SHARED((NP, D), jnp.float32),
        pltpu.SemaphoreType.DMA,
        pltpu.SemaphoreType.DMA,
        pltpu.SemaphoreType.DMA,
        pltpu.SemaphoreType.DMA,
    ],
)


def _split_trans(res, tr_ref):
    for t in range(T):
        tr_ref[t] = res[:, t * D:(t + 1) * D]


def _trans_body(h_ref, wcat_ref, bcat_ref, tr_ref):
    res = jnp.dot(h_ref[...], wcat_ref[...],
                  preferred_element_type=jnp.float32) + bcat_ref[...]
    _split_trans(res, tr_ref)


_k_trans = pl.pallas_call(
    _trans_body,
    grid=(NP // BR,),
    in_specs=[
        pl.BlockSpec((BR, D), lambda i: (i, 0)),
        pl.BlockSpec((D, T * D), lambda i: (0, 0)),
        pl.BlockSpec((1, T * D), lambda i: (0, 0)),
    ],
    out_specs=pl.BlockSpec((T, BR, D), lambda i: (0, i, 0)),
    out_shape=jax.ShapeDtypeStruct((T, NP, D), jnp.float32),
)


def _gru_core(a0_ref, a1_ref, h_ref, wih_ref, whh_ref, bih_ref, bhh_ref):
    a = a0_ref[...] + a1_ref[...]
    h = h_ref[...]
    gi = jnp.dot(a, wih_ref[...], preferred_element_type=jnp.float32) + bih_ref[...]
    gh = jnp.dot(h, whh_ref[...], preferred_element_type=jnp.float32) + bhh_ref[...]
    r = jax.nn.sigmoid(gi[:, :D] + gh[:, :D])
    z = jax.nn.sigmoid(gi[:, D:2 * D] + gh[:, D:2 * D])
    n = jnp.tanh(gi[:, 2 * D:] + r * gh[:, 2 * D:])
    return (1.0 - z) * n + z * h


def _gru_trans_body(a0_ref, a1_ref, h_ref, wih_ref, whh_ref, bih_ref,
                    bhh_ref, wcat_ref, bcat_ref, hn_ref, tr_ref):
    hn = _gru_core(a0_ref, a1_ref, h_ref, wih_ref, whh_ref, bih_ref, bhh_ref)
    hn_ref[...] = hn
    res = jnp.dot(hn, wcat_ref[...],
                  preferred_element_type=jnp.float32) + bcat_ref[...]
    _split_trans(res, tr_ref)


_k_gru_trans = pl.pallas_call(
    _gru_trans_body,
    grid=(NP // BR,),
    in_specs=[
        pl.BlockSpec((BR, D), lambda i: (i, 0)),
        pl.BlockSpec((BR, D), lambda i: (i, 0)),
        pl.BlockSpec((BR, D), lambda i: (i, 0)),
        pl.BlockSpec((D, 3 * D), lambda i: (0, 0)),
        pl.BlockSpec((D, 3 * D), lambda i: (0, 0)),
        pl.BlockSpec((1, 3 * D), lambda i: (0, 0)),
        pl.BlockSpec((1, 3 * D), lambda i: (0, 0)),
        pl.BlockSpec((D, T * D), lambda i: (0, 0)),
        pl.BlockSpec((1, T * D), lambda i: (0, 0)),
    ],
    out_specs=[
        pl.BlockSpec((BR, D), lambda i: (i, 0)),
        pl.BlockSpec((T, BR, D), lambda i: (0, i, 0)),
    ],
    out_shape=[
        jax.ShapeDtypeStruct((NP, D), jnp.float32),
        jax.ShapeDtypeStruct((T, NP, D), jnp.float32),
    ],
)


def _gru_body(a0_ref, a1_ref, h_ref, wih_ref, whh_ref, bih_ref, bhh_ref,
              hn_ref):
    hn_ref[...] = _gru_core(a0_ref, a1_ref, h_ref, wih_ref, whh_ref, bih_ref,
                            bhh_ref)


_k_gru = pl.pallas_call(
    _gru_body,
    grid=(NP // BR,),
    in_specs=[
        pl.BlockSpec((BR, D), lambda i: (i, 0)),
        pl.BlockSpec((BR, D), lambda i: (i, 0)),
        pl.BlockSpec((BR, D), lambda i: (i, 0)),
        pl.BlockSpec((D, 3 * D), lambda i: (0, 0)),
        pl.BlockSpec((D, 3 * D), lambda i: (0, 0)),
        pl.BlockSpec((1, 3 * D), lambda i: (0, 0)),
        pl.BlockSpec((1, 3 * D), lambda i: (0, 0)),
    ],
    out_specs=pl.BlockSpec((BR, D), lambda i: (i, 0)),
    out_shape=jax.ShapeDtypeStruct((NP, D), jnp.float32),
)


def _pool_body(h_ref, ann_ref, gwh_ref, gwa_ref, gb_ref, owh_ref, owa_ref,
               ob_ref, out_ref):
    h = h_ref[...]
    ann = ann_ref[...]
    g = (jnp.sum(h * gwh_ref[...], axis=1, keepdims=True)
         + jnp.sum(ann * gwa_ref[...], axis=1, keepdims=True) + gb_ref[0, 0])
    row = lax.broadcasted_iota(jnp.int32, (NP, 1), 0)
    g = jnp.where(row < N, g, -jnp.inf)
    m = jnp.max(g)
    w = jnp.exp(g - m)
    sw = jnp.sum(w)
    sh = jnp.sum(w * h, axis=0, keepdims=True)
    sa = jnp.sum(w * ann, axis=0, keepdims=True)
    logits = (jnp.dot(sh, owh_ref[...], preferred_element_type=jnp.float32)
              + jnp.dot(sa, owa_ref[...], preferred_element_type=jnp.float32))
    out_ref[...] = logits / sw + ob_ref[...]


_k_pool = pl.pallas_call(
    _pool_body,
    out_shape=jax.ShapeDtypeStruct((1, 128), jnp.float32),
)


@jax.jit
def _run(annotation, edge_index, etypes, W_et, b_et, w_ih, w_hh, b_ih, b_hh,
         gate_w, gate_b, out_w, out_b):
    h0 = jnp.zeros((NP, D), jnp.float32).at[:N, :ANN].set(annotation)
    ann_p = jnp.zeros((NP, ANN), jnp.float32).at[:N].set(annotation)
    src = edge_index[0].astype(jnp.int32)
    dst = edge_index[1].astype(jnp.int32)
    gidx = etypes.astype(jnp.int32) * NP + src
    pad_ids = jnp.arange(EP - E, dtype=jnp.int32)
    gidx_p = jnp.concatenate([gidx, pad_ids % N]).reshape(NW, 2, HC, CHUNK)
    dst_p = jnp.concatenate([dst, N + pad_ids % (NP - N)]).reshape(
        NW, 2, 2 * HC, CHUNK // 2)

    wcat = jnp.transpose(W_et, (2, 0, 1)).reshape(D, T * D)
    bcat = b_et.reshape(1, T * D)
    wih_t = w_ih.T
    whh_t = w_hh.T
    bih = b_ih.reshape(1, 3 * D)
    bhh = b_hh.reshape(1, 3 * D)
    gwh = gate_w[:, :D]
    gwa = gate_w[:, D:]
    gb = gate_b.reshape(1, 1)
    owh = jnp.zeros((D, 128), jnp.float32).at[:, :CLS].set(out_w[:, :D].T)
    owa = jnp.zeros((ANN, 128), jnp.float32).at[:, :CLS].set(out_w[:, D:].T)
    ob = jnp.zeros((1, 128), jnp.float32).at[0, :CLS].set(out_b)

    h = h0
    trans = _k_trans(h, wcat, bcat)
    for step in range(STEPS):
        parts = _sc_scatter(trans.reshape(T * NP, D), gidx_p, dst_p)
        if step < STEPS - 1:
            h, trans = _k_gru_trans(parts[0], parts[1], h, wih_t, whh_t,
                                    bih, bhh, wcat, bcat)
        else:
            h = _k_gru(parts[0], parts[1], h, wih_t, whh_t, bih, bhh)

    logits = _k_pool(h, ann_p, gwh, gwa, gb, owh, owa, ob)
    return logits[:, :CLS]


def kernel(annotation, edge_index, etypes, W_et, b_et, w_ih, w_hh, b_ih, b_hh,
           gate_w, gate_b, out_w, out_b):
    return _run(annotation, edge_index, etypes, W_et, b_et, w_ih, w_hh, b_ih,
                b_hh, gate_w, gate_b, out_w, out_b)


# speedup vs baseline: 1.1478x; 1.1478x over previous
import jax
import jax.numpy as jnp
from jax import lax
from jax.experimental import pallas as pl
from jax.experimental.pallas import tpu as pltpu
from jax.experimental.pallas import tpu_sc as plsc

N = 10000
E = 320000
ANN = 64
D = 128
T = 4
STEPS = 5
CLS = 10

NP = 10240
NSUB = 16
NW = 32
RT = NP // NSUB
CHUNK = 128
HC = 40
CT = 2 * HC
EW = CT * CHUNK
EP = NW * EW
BR = 1280


def _sc_body(trans_hbm, gidx_hbm, dst_hbm, out0_hbm, out1_hbm,
             gidx_v, dst_v, rows_a, rows_b, acc, sem_a, sem_b):
    c = lax.axis_index("c")
    s = lax.axis_index("s")
    wid = c * NSUB + s

    def zrow(r, carry):
        for k in range(D // 16):
            rows_a[r, pl.ds(k * 16, 16)] = jnp.zeros((16,), jnp.float32)
        return carry
    lax.fori_loop(0, CHUNK, zrow, 0)
    for i in range(RT // CHUNK):
        pltpu.sync_copy(rows_a, acc.at[pl.ds(s * RT + i * CHUNK, CHUNK)])
    plsc.subcore_barrier()

    def half(hb, carry):
        pltpu.sync_copy(gidx_hbm.at[wid, hb], gidx_v)
        pltpu.sync_copy(dst_hbm.at[wid, hb], dst_v)
        pltpu.async_copy(trans_hbm.at[gidx_v.at[0]], rows_a, sem_a)

        def pair(jj, carry2):
            j0 = jj * 2
            pltpu.make_async_copy(trans_hbm.at[gidx_v.at[j0]], rows_a,
                                  sem_a).wait()
            pltpu.async_copy(trans_hbm.at[gidx_v.at[j0 + 1]], rows_b, sem_b)
            pltpu.sync_copy(rows_a, acc.at[dst_v.at[j0]], add=True)
            pltpu.make_async_copy(trans_hbm.at[gidx_v.at[j0 + 1]], rows_b,
                                  sem_b).wait()
            pltpu.async_copy(trans_hbm.at[gidx_v.at[j0 + 2]], rows_a, sem_a)
            pltpu.sync_copy(rows_b, acc.at[dst_v.at[j0 + 1]], add=True)
            return carry2
        lax.fori_loop(0, HC // 2 - 1, pair, 0)

        j0 = HC - 2
        pltpu.make_async_copy(trans_hbm.at[gidx_v.at[j0]], rows_a,
                              sem_a).wait()
        pltpu.async_copy(trans_hbm.at[gidx_v.at[j0 + 1]], rows_b, sem_b)
        pltpu.sync_copy(rows_a, acc.at[dst_v.at[j0]], add=True)
        pltpu.make_async_copy(trans_hbm.at[gidx_v.at[j0 + 1]], rows_b,
                              sem_b).wait()
        pltpu.sync_copy(rows_b, acc.at[dst_v.at[j0 + 1]], add=True)
        return carry
    lax.fori_loop(0, 2, half, 0)
    plsc.subcore_barrier()

    @pl.when(c == 0)
    def _():
        pltpu.sync_copy(acc.at[pl.ds(s * RT, RT)], out0_hbm.at[pl.ds(s * RT, RT)])

    @pl.when(c == 1)
    def _():
        pltpu.sync_copy(acc.at[pl.ds(s * RT, RT)], out1_hbm.at[pl.ds(s * RT, RT)])


_sc_scatter = pl.kernel(
    _sc_body,
    out_type=(jax.ShapeDtypeStruct((NP, D), jnp.float32),
              jax.ShapeDtypeStruct((NP, D), jnp.float32)),
    mesh=plsc.VectorSubcoreMesh(core_axis_name="c", subcore_axis_name="s"),
    scratch_types=[
        pltpu.VMEM((HC, CHUNK), jnp.int32),
        pltpu.VMEM((HC, CHUNK), jnp.int32),
        pltpu.VMEM((CHUNK, D), jnp.float32),
        pltpu.VMEM((CHUNK, D), jnp.float32),
        pltpu.VMEM_SHARED((NP, D), jnp.float32),
        pltpu.SemaphoreType.DMA,
        pltpu.SemaphoreType.DMA,
    ],
)


def _split_trans(res, tr_ref):
    for t in range(T):
        tr_ref[t] = res[:, t * D:(t + 1) * D]


def _trans_body(h_ref, wcat_ref, bcat_ref, tr_ref):
    res = jnp.dot(h_ref[...], wcat_ref[...],
                  preferred_element_type=jnp.float32) + bcat_ref[...]
    _split_trans(res, tr_ref)


_k_trans = pl.pallas_call(
    _trans_body,
    grid=(NP // BR,),
    in_specs=[
        pl.BlockSpec((BR, D), lambda i: (i, 0)),
        pl.BlockSpec((D, T * D), lambda i: (0, 0)),
        pl.BlockSpec((1, T * D), lambda i: (0, 0)),
    ],
    out_specs=pl.BlockSpec((T, BR, D), lambda i: (0, i, 0)),
    out_shape=jax.ShapeDtypeStruct((T, NP, D), jnp.float32),
)


def _gru_core(a0_ref, a1_ref, h_ref, wih_ref, whh_ref, bih_ref, bhh_ref):
    a = a0_ref[...] + a1_ref[...]
    h = h_ref[...]
    gi = jnp.dot(a, wih_ref[...], preferred_element_type=jnp.float32) + bih_ref[...]
    gh = jnp.dot(h, whh_ref[...], preferred_element_type=jnp.float32) + bhh_ref[...]
    r = jax.nn.sigmoid(gi[:, :D] + gh[:, :D])
    z = jax.nn.sigmoid(gi[:, D:2 * D] + gh[:, D:2 * D])
    n = jnp.tanh(gi[:, 2 * D:] + r * gh[:, 2 * D:])
    return (1.0 - z) * n + z * h


def _gru_trans_body(a0_ref, a1_ref, h_ref, wih_ref, whh_ref, bih_ref,
                    bhh_ref, wcat_ref, bcat_ref, hn_ref, tr_ref):
    hn = _gru_core(a0_ref, a1_ref, h_ref, wih_ref, whh_ref, bih_ref, bhh_ref)
    hn_ref[...] = hn
    res = jnp.dot(hn, wcat_ref[...],
                  preferred_element_type=jnp.float32) + bcat_ref[...]
    _split_trans(res, tr_ref)


_k_gru_trans = pl.pallas_call(
    _gru_trans_body,
    grid=(NP // BR,),
    in_specs=[
        pl.BlockSpec((BR, D), lambda i: (i, 0)),
        pl.BlockSpec((BR, D), lambda i: (i, 0)),
        pl.BlockSpec((BR, D), lambda i: (i, 0)),
        pl.BlockSpec((D, 3 * D), lambda i: (0, 0)),
        pl.BlockSpec((D, 3 * D), lambda i: (0, 0)),
        pl.BlockSpec((1, 3 * D), lambda i: (0, 0)),
        pl.BlockSpec((1, 3 * D), lambda i: (0, 0)),
        pl.BlockSpec((D, T * D), lambda i: (0, 0)),
        pl.BlockSpec((1, T * D), lambda i: (0, 0)),
    ],
    out_specs=[
        pl.BlockSpec((BR, D), lambda i: (i, 0)),
        pl.BlockSpec((T, BR, D), lambda i: (0, i, 0)),
    ],
    out_shape=[
        jax.ShapeDtypeStruct((NP, D), jnp.float32),
        jax.ShapeDtypeStruct((T, NP, D), jnp.float32),
    ],
)


def _gru_body(a0_ref, a1_ref, h_ref, wih_ref, whh_ref, bih_ref, bhh_ref,
              hn_ref):
    hn_ref[...] = _gru_core(a0_ref, a1_ref, h_ref, wih_ref, whh_ref, bih_ref,
                            bhh_ref)


_k_gru = pl.pallas_call(
    _gru_body,
    grid=(NP // BR,),
    in_specs=[
        pl.BlockSpec((BR, D), lambda i: (i, 0)),
        pl.BlockSpec((BR, D), lambda i: (i, 0)),
        pl.BlockSpec((BR, D), lambda i: (i, 0)),
        pl.BlockSpec((D, 3 * D), lambda i: (0, 0)),
        pl.BlockSpec((D, 3 * D), lambda i: (0, 0)),
        pl.BlockSpec((1, 3 * D), lambda i: (0, 0)),
        pl.BlockSpec((1, 3 * D), lambda i: (0, 0)),
    ],
    out_specs=pl.BlockSpec((BR, D), lambda i: (i, 0)),
    out_shape=jax.ShapeDtypeStruct((NP, D), jnp.float32),
)


def _pool_body(h_ref, ann_ref, gwh_ref, gwa_ref, gb_ref, owh_ref, owa_ref,
               ob_ref, out_ref):
    h = h_ref[...]
    ann = ann_ref[...]
    g = (jnp.sum(h * gwh_ref[...], axis=1, keepdims=True)
         + jnp.sum(ann * gwa_ref[...], axis=1, keepdims=True) + gb_ref[0, 0])
    row = lax.broadcasted_iota(jnp.int32, (NP, 1), 0)
    g = jnp.where(row < N, g, -jnp.inf)
    m = jnp.max(g)
    w = jnp.exp(g - m)
    sw = jnp.sum(w)
    sh = jnp.sum(w * h, axis=0, keepdims=True)
    sa = jnp.sum(w * ann, axis=0, keepdims=True)
    logits = (jnp.dot(sh, owh_ref[...], preferred_element_type=jnp.float32)
              + jnp.dot(sa, owa_ref[...], preferred_element_type=jnp.float32))
    out_ref[...] = logits / sw + ob_ref[...]


_k_pool = pl.pallas_call(
    _pool_body,
    out_shape=jax.ShapeDtypeStruct((1, 128), jnp.float32),
)


@jax.jit
def _run(annotation, edge_index, etypes, W_et, b_et, w_ih, w_hh, b_ih, b_hh,
         gate_w, gate_b, out_w, out_b):
    h0 = jnp.zeros((NP, D), jnp.float32).at[:N, :ANN].set(annotation)
    ann_p = jnp.zeros((NP, ANN), jnp.float32).at[:N].set(annotation)
    src = edge_index[0].astype(jnp.int32)
    dst = edge_index[1].astype(jnp.int32)
    gidx = etypes.astype(jnp.int32) * NP + src
    pad_ids = jnp.arange(EP - E, dtype=jnp.int32)
    gidx_p = jnp.concatenate([gidx, pad_ids % N]).reshape(NW, 2, HC, CHUNK)
    dst_p = jnp.concatenate([dst, N + pad_ids % (NP - N)]).reshape(
        NW, 2, HC, CHUNK)

    wcat = jnp.transpose(W_et, (2, 0, 1)).reshape(D, T * D)
    bcat = b_et.reshape(1, T * D)
    wih_t = w_ih.T
    whh_t = w_hh.T
    bih = b_ih.reshape(1, 3 * D)
    bhh = b_hh.reshape(1, 3 * D)
    gwh = gate_w[:, :D]
    gwa = gate_w[:, D:]
    gb = gate_b.reshape(1, 1)
    owh = jnp.zeros((D, 128), jnp.float32).at[:, :CLS].set(out_w[:, :D].T)
    owa = jnp.zeros((ANN, 128), jnp.float32).at[:, :CLS].set(out_w[:, D:].T)
    ob = jnp.zeros((1, 128), jnp.float32).at[0, :CLS].set(out_b)

    h = h0
    trans = _k_trans(h, wcat, bcat)
    for step in range(STEPS):
        parts = _sc_scatter(trans.reshape(T * NP, D), gidx_p, dst_p)
        if step < STEPS - 1:
            h, trans = _k_gru_trans(parts[0], parts[1], h, wih_t, whh_t,
                                    bih, bhh, wcat, bcat)
        else:
            h = _k_gru(parts[0], parts[1], h, wih_t, whh_t, bih, bhh)

    logits = _k_pool(h, ann_p, gwh, gwa, gb, owh, owa, ob)
    return logits[:, :CLS]


def kernel(annotation, edge_index, etypes, W_et, b_et, w_ih, w_hh, b_ih, b_hh,
           gate_w, gate_b, out_w, out_b):
    return _run(annotation, edge_index, etypes, W_et, b_et, w_ih, w_hh, b_ih,
                b_hh, gate_w, gate_b, out_w, out_b)


# speedup vs baseline: 1.1501x; 1.0020x over previous
import jax
import jax.numpy as jnp
from jax import lax
from jax.experimental import pallas as pl
from jax.experimental.pallas import tpu as pltpu
from jax.experimental.pallas import tpu_sc as plsc

N = 10000
E = 320000
ANN = 64
D = 128
T = 4
STEPS = 5
CLS = 10

NP = 10240
NSUB = 16
NW = 32
RT = NP // NSUB
CHUNK = 128
HC = 40
CT = 2 * HC
EW = CT * CHUNK
EP = NW * EW
BR = 2048


def _sc_body(trans_hbm, gidx_hbm, dst_hbm, out0_hbm, out1_hbm,
             gidx_v, dst_v, rows_a, rows_b, acc, sem_a, sem_b):
    c = lax.axis_index("c")
    s = lax.axis_index("s")
    wid = c * NSUB + s

    def zrow(r, carry):
        for k in range(D // 16):
            rows_a[r, pl.ds(k * 16, 16)] = jnp.zeros((16,), jnp.float32)
        return carry
    lax.fori_loop(0, CHUNK, zrow, 0)
    for i in range(RT // CHUNK):
        pltpu.sync_copy(rows_a, acc.at[pl.ds(s * RT + i * CHUNK, CHUNK)])
    plsc.subcore_barrier()

    def half(hb, carry):
        pltpu.sync_copy(gidx_hbm.at[wid, hb], gidx_v)
        pltpu.sync_copy(dst_hbm.at[wid, hb], dst_v)
        pltpu.async_copy(trans_hbm.at[gidx_v.at[0]], rows_a, sem_a)

        def pair(jj, carry2):
            j0 = jj * 2
            pltpu.make_async_copy(trans_hbm.at[gidx_v.at[j0]], rows_a,
                                  sem_a).wait()
            pltpu.async_copy(trans_hbm.at[gidx_v.at[j0 + 1]], rows_b, sem_b)
            pltpu.sync_copy(rows_a, acc.at[dst_v.at[j0]], add=True)
            pltpu.make_async_copy(trans_hbm.at[gidx_v.at[j0 + 1]], rows_b,
                                  sem_b).wait()
            pltpu.async_copy(trans_hbm.at[gidx_v.at[j0 + 2]], rows_a, sem_a)
            pltpu.sync_copy(rows_b, acc.at[dst_v.at[j0 + 1]], add=True)
            return carry2
        lax.fori_loop(0, HC // 2 - 1, pair, 0)

        j0 = HC - 2
        pltpu.make_async_copy(trans_hbm.at[gidx_v.at[j0]], rows_a,
                              sem_a).wait()
        pltpu.async_copy(trans_hbm.at[gidx_v.at[j0 + 1]], rows_b, sem_b)
        pltpu.sync_copy(rows_a, acc.at[dst_v.at[j0]], add=True)
        pltpu.make_async_copy(trans_hbm.at[gidx_v.at[j0 + 1]], rows_b,
                              sem_b).wait()
        pltpu.sync_copy(rows_b, acc.at[dst_v.at[j0 + 1]], add=True)
        return carry
    lax.fori_loop(0, 2, half, 0)
    plsc.subcore_barrier()

    @pl.when(c == 0)
    def _():
        pltpu.sync_copy(acc.at[pl.ds(s * RT, RT)], out0_hbm.at[pl.ds(s * RT, RT)])

    @pl.when(c == 1)
    def _():
        pltpu.sync_copy(acc.at[pl.ds(s * RT, RT)], out1_hbm.at[pl.ds(s * RT, RT)])


_sc_scatter = pl.kernel(
    _sc_body,
    out_type=(jax.ShapeDtypeStruct((NP, D), jnp.float32),
              jax.ShapeDtypeStruct((NP, D), jnp.float32)),
    mesh=plsc.VectorSubcoreMesh(core_axis_name="c", subcore_axis_name="s"),
    scratch_types=[
        pltpu.VMEM((HC, CHUNK), jnp.int32),
        pltpu.VMEM((HC, CHUNK), jnp.int32),
        pltpu.VMEM((CHUNK, D), jnp.float32),
        pltpu.VMEM((CHUNK, D), jnp.float32),
        pltpu.VMEM_SHARED((NP, D), jnp.float32),
        pltpu.SemaphoreType.DMA,
        pltpu.SemaphoreType.DMA,
    ],
)


def _split_trans(res, tr_ref):
    for t in range(T):
        tr_ref[t] = res[:, t * D:(t + 1) * D]


def _trans_body(h_ref, wcat_ref, bcat_ref, tr_ref):
    res = jnp.dot(h_ref[...], wcat_ref[...],
                  preferred_element_type=jnp.float32) + bcat_ref[...]
    _split_trans(res, tr_ref)


_k_trans = pl.pallas_call(
    _trans_body,
    grid=(NP // BR,),
    in_specs=[
        pl.BlockSpec((BR, D), lambda i: (i, 0)),
        pl.BlockSpec((D, T * D), lambda i: (0, 0)),
        pl.BlockSpec((1, T * D), lambda i: (0, 0)),
    ],
    out_specs=pl.BlockSpec((T, BR, D), lambda i: (0, i, 0)),
    out_shape=jax.ShapeDtypeStruct((T, NP, D), jnp.float32),
)


def _gru_core(a0_ref, a1_ref, h_ref, wih_ref, whh_ref, bih_ref, bhh_ref):
    a = a0_ref[...] + a1_ref[...]
    h = h_ref[...]
    gi = jnp.dot(a, wih_ref[...], preferred_element_type=jnp.float32) + bih_ref[...]
    gh = jnp.dot(h, whh_ref[...], preferred_element_type=jnp.float32) + bhh_ref[...]
    r = jax.nn.sigmoid(gi[:, :D] + gh[:, :D])
    z = jax.nn.sigmoid(gi[:, D:2 * D] + gh[:, D:2 * D])
    n = jnp.tanh(gi[:, 2 * D:] + r * gh[:, 2 * D:])
    return (1.0 - z) * n + z * h


def _gru_trans_body(a0_ref, a1_ref, h_ref, wih_ref, whh_ref, bih_ref,
                    bhh_ref, wcat_ref, bcat_ref, hn_ref, tr_ref):
    hn = _gru_core(a0_ref, a1_ref, h_ref, wih_ref, whh_ref, bih_ref, bhh_ref)
    hn_ref[...] = hn
    res = jnp.dot(hn, wcat_ref[...],
                  preferred_element_type=jnp.float32) + bcat_ref[...]
    _split_trans(res, tr_ref)


_k_gru_trans = pl.pallas_call(
    _gru_trans_body,
    grid=(NP // BR,),
    in_specs=[
        pl.BlockSpec((BR, D), lambda i: (i, 0)),
        pl.BlockSpec((BR, D), lambda i: (i, 0)),
        pl.BlockSpec((BR, D), lambda i: (i, 0)),
        pl.BlockSpec((D, 3 * D), lambda i: (0, 0)),
        pl.BlockSpec((D, 3 * D), lambda i: (0, 0)),
        pl.BlockSpec((1, 3 * D), lambda i: (0, 0)),
        pl.BlockSpec((1, 3 * D), lambda i: (0, 0)),
        pl.BlockSpec((D, T * D), lambda i: (0, 0)),
        pl.BlockSpec((1, T * D), lambda i: (0, 0)),
    ],
    out_specs=[
        pl.BlockSpec((BR, D), lambda i: (i, 0)),
        pl.BlockSpec((T, BR, D), lambda i: (0, i, 0)),
    ],
    out_shape=[
        jax.ShapeDtypeStruct((NP, D), jnp.float32),
        jax.ShapeDtypeStruct((T, NP, D), jnp.float32),
    ],
)


def _gru_body(a0_ref, a1_ref, h_ref, wih_ref, whh_ref, bih_ref, bhh_ref,
              hn_ref):
    hn_ref[...] = _gru_core(a0_ref, a1_ref, h_ref, wih_ref, whh_ref, bih_ref,
                            bhh_ref)


_k_gru = pl.pallas_call(
    _gru_body,
    grid=(NP // BR,),
    in_specs=[
        pl.BlockSpec((BR, D), lambda i: (i, 0)),
        pl.BlockSpec((BR, D), lambda i: (i, 0)),
        pl.BlockSpec((BR, D), lambda i: (i, 0)),
        pl.BlockSpec((D, 3 * D), lambda i: (0, 0)),
        pl.BlockSpec((D, 3 * D), lambda i: (0, 0)),
        pl.BlockSpec((1, 3 * D), lambda i: (0, 0)),
        pl.BlockSpec((1, 3 * D), lambda i: (0, 0)),
    ],
    out_specs=pl.BlockSpec((BR, D), lambda i: (i, 0)),
    out_shape=jax.ShapeDtypeStruct((NP, D), jnp.float32),
)


def _pool_body(h_ref, ann_ref, gwh_ref, gwa_ref, gb_ref, owh_ref, owa_ref,
               ob_ref, out_ref):
    h = h_ref[...]
    ann = ann_ref[...]
    g = (jnp.sum(h * gwh_ref[...], axis=1, keepdims=True)
         + jnp.sum(ann * gwa_ref[...], axis=1, keepdims=True) + gb_ref[0, 0])
    row = lax.broadcasted_iota(jnp.int32, (NP, 1), 0)
    g = jnp.where(row < N, g, -jnp.inf)
    m = jnp.max(g)
    w = jnp.exp(g - m)
    sw = jnp.sum(w)
    sh = jnp.sum(w * h, axis=0, keepdims=True)
    sa = jnp.sum(w * ann, axis=0, keepdims=True)
    logits = (jnp.dot(sh, owh_ref[...], preferred_element_type=jnp.float32)
              + jnp.dot(sa, owa_ref[...], preferred_element_type=jnp.float32))
    out_ref[...] = logits / sw + ob_ref[...]


_k_pool = pl.pallas_call(
    _pool_body,
    out_shape=jax.ShapeDtypeStruct((1, 128), jnp.float32),
)


@jax.jit
def _run(annotation, edge_index, etypes, W_et, b_et, w_ih, w_hh, b_ih, b_hh,
         gate_w, gate_b, out_w, out_b):
    h0 = jnp.zeros((NP, D), jnp.float32).at[:N, :ANN].set(annotation)
    ann_p = jnp.zeros((NP, ANN), jnp.float32).at[:N].set(annotation)
    src = edge_index[0].astype(jnp.int32)
    dst = edge_index[1].astype(jnp.int32)
    gidx = etypes.astype(jnp.int32) * NP + src
    pad_ids = jnp.arange(EP - E, dtype=jnp.int32)
    gidx_p = jnp.concatenate([gidx, pad_ids % N]).reshape(NW, 2, HC, CHUNK)
    dst_p = jnp.concatenate([dst, N + pad_ids % (NP - N)]).reshape(
        NW, 2, HC, CHUNK)

    wcat = jnp.transpose(W_et, (2, 0, 1)).reshape(D, T * D)
    bcat = b_et.reshape(1, T * D)
    wih_t = w_ih.T
    whh_t = w_hh.T
    bih = b_ih.reshape(1, 3 * D)
    bhh = b_hh.reshape(1, 3 * D)
    gwh = gate_w[:, :D]
    gwa = gate_w[:, D:]
    gb = gate_b.reshape(1, 1)
    owh = jnp.zeros((D, 128), jnp.float32).at[:, :CLS].set(out_w[:, :D].T)
    owa = jnp.zeros((ANN, 128), jnp.float32).at[:, :CLS].set(out_w[:, D:].T)
    ob = jnp.zeros((1, 128), jnp.float32).at[0, :CLS].set(out_b)

    h = h0
    trans = _k_trans(h, wcat, bcat)
    for step in range(STEPS):
        parts = _sc_scatter(trans.reshape(T * NP, D), gidx_p, dst_p)
        if step < STEPS - 1:
            h, trans = _k_gru_trans(parts[0], parts[1], h, wih_t, whh_t,
                                    bih, bhh, wcat, bcat)
        else:
            h = _k_gru(parts[0], parts[1], h, wih_t, whh_t, bih, bhh)

    logits = _k_pool(h, ann_p, gwh, gwa, gb, owh, owa, ob)
    return logits[:, :CLS]


def kernel(annotation, edge_index, etypes, W_et, b_et, w_ih, w_hh, b_ih, b_hh,
           gate_w, gate_b, out_w, out_b):
    return _run(annotation, edge_index, etypes, W_et, b_et, w_ih, w_hh, b_ih,
                b_hh, gate_w, gate_b, out_w, out_b)
